# baseline (speedup 1.0000x reference)
import functools

import jax
import jax.numpy as jnp
from jax import lax
from jax.experimental import pallas as pl
from jax.experimental.pallas import tpu as pltpu
from jax.experimental.pallas import tpu_sc as plsc

_E = 64
_N_GROUPS = 8
_GS = _E // _N_GROUPS
_TOPK_GROUPS = 4
_TOPK = 8
_ROUTE_SCALE = 2.5

_NC = 2
_NS = 16
_NW = _NC * _NS
_L = 16


def _scores_body(x_ref, w_ref, b_ref, out_ref):
    logits = jax.lax.dot_general(
        w_ref[...], x_ref[...], dimension_numbers=(((1,), (1,)), ((), ())),
        preferred_element_type=jnp.float32)
    out_ref[...] = jax.nn.sigmoid(logits) + b_ref[...]


def _scores_t(x, w_pad, bias_col, t_half, blk_off):
    dim = x.shape[1]
    tt = 1024
    return pl.pallas_call(
        _scores_body,
        grid=(t_half // tt,),
        in_specs=[
            pl.BlockSpec((tt, dim), lambda i: (i + blk_off, 0)),
            pl.BlockSpec((_E, dim), lambda i: (0, 0)),
            pl.BlockSpec((_E, 1), lambda i: (0, 0)),
        ],
        out_specs=pl.BlockSpec((_E, tt), lambda i: (0, i)),
        out_shape=jax.ShapeDtypeStruct((_E, t_half), jnp.float32),
    )(x, w_pad, bias_col)


def _tree(op, xs):
    xs = list(xs)
    while len(xs) > 1:
        nxt = [op(xs[i], xs[i + 1]) for i in range(0, len(xs) - 1, 2)]
        if len(xs) % 2:
            nxt.append(xs[-1])
        xs = nxt
    return xs[0]


def _top2_sum(vs):
    neg = jnp.full((_L,), -jnp.inf, jnp.float32)
    pairs = [(x, neg) for x in vs]
    while len(pairs) > 1:
        nxt = []
        for i in range(0, len(pairs) - 1, 2):
            (a1, a2), (b1, b2) = pairs[i], pairs[i + 1]
            m1 = jnp.maximum(a1, b1)
            m2 = jnp.maximum(jnp.minimum(a1, b1), jnp.maximum(a2, b2))
            nxt.append((m1, m2))
        if len(pairs) % 2:
            nxt.append(pairs[-1])
        pairs = nxt
    m1, m2 = pairs[0]
    return m1 + m2


def _route_chunk(v, bias_ref):
    neg = jnp.full((_L,), -jnp.inf, jnp.float32)
    one = jnp.full((_L,), 1.0, jnp.float32)
    zero = jnp.full((_L,), 0.0, jnp.float32)

    t2 = [_top2_sum(v[g * _GS:(g + 1) * _GS]) for g in range(_N_GROUPS)]

    sel = []
    for g in range(_N_GROUPS):
        terms = []
        for h in range(_N_GROUPS):
            if h == g:
                continue
            c = (t2[h] >= t2[g]) if h < g else (t2[h] > t2[g])
            terms.append(jnp.where(c, one, zero))
        rk = _tree(jnp.add, terms)
        sel.append(rk < float(_TOPK_GROUPS))

    sm = [jnp.where(sel[e // _GS], v[e], neg) for e in range(_E)]
    e_const = [jnp.full((_L,), e, jnp.int32) for e in range(_E)]
    big = jnp.full((_L,), _E, jnp.int32)

    widx, wraw = [], []
    for r in range(_TOPK):
        m = _tree(jnp.maximum, sm)
        cand = [jnp.where(sm[e] == m, e_const[e], big) for e in range(_E)]
        idx = _tree(jnp.minimum, cand)
        bias_at = plsc.load_gather(bias_ref, [idx])
        widx.append(idx)
        wraw.append(m - bias_at)
        if r < _TOPK - 1:
            sm = [jnp.where(cand[e] == idx, neg, sm[e]) for e in range(_E)]

    wsum = _tree(jnp.add, wraw)
    wvals = [(w / wsum) * _ROUTE_SCALE for w in wraw]
    return wvals, widx


def _route_body(sT, bias_hbm, wout, iout,
                span_v, wspan_v, ispan_v, bias_v,
                *, span):
    wid = lax.axis_index("s") * _NC + lax.axis_index("c")
    base = wid * span
    pltpu.sync_copy(bias_hbm, bias_v)
    pltpu.sync_copy(sT.at[:, pl.ds(base, span)], span_v)
    def chunk(c, carry):
        off = c * _L
        v = [span_v[e, pl.ds(off, _L)] for e in range(_E)]
        wvals, ivals = _route_chunk(v, bias_v)
        for r in range(_TOPK):
            wspan_v[r, pl.ds(off, _L)] = wvals[r]
            ispan_v[r, pl.ds(off, _L)] = ivals[r]
        return carry

    lax.fori_loop(0, span // _L, chunk, 0)
    pltpu.sync_copy(wspan_v, wout.at[:, pl.ds(base, span)])
    pltpu.sync_copy(ispan_v, iout.at[:, pl.ds(base, span)])


def _route(s_t, bias):
    t = s_t.shape[1]
    span = t // _NW
    mesh = plsc.VectorSubcoreMesh(
        core_axis_name="c", subcore_axis_name="s",
        num_cores=_NC, num_subcores=_NS)
    fn = pl.kernel(
        functools.partial(_route_body, span=span),
        out_type=[
            jax.ShapeDtypeStruct((_TOPK, t), jnp.float32),
            jax.ShapeDtypeStruct((_TOPK, t), jnp.int32),
        ],
        mesh=mesh,
        compiler_params=pltpu.CompilerParams(
            use_tc_tiling_on_sc=False, needs_layout_passes=False),
        scratch_types=[
            pltpu.VMEM((_E, span), jnp.float32),
            pltpu.VMEM((_TOPK, span), jnp.float32),
            pltpu.VMEM((_TOPK, span), jnp.int32),
            pltpu.VMEM((_E,), jnp.float32),
        ],
    )
    return fn(s_t, bias)


_N_SPLITS = 2


@jax.jit
def kernel(x, weight, bias):
    t = x.shape[0]
    e = weight.shape[0]
    w_pad = jnp.pad(weight, ((0, 0), (1, 0)))
    bias_col = bias.reshape(e, 1).astype(jnp.float32)
    bias_f = bias.astype(jnp.float32)

    t_half = t // _N_SPLITS
    tt = 1024
    w_parts, i_parts = [], []
    for k in range(_N_SPLITS):
        s_k = _scores_t(x, w_pad, bias_col, t_half, k * (t_half // tt))
        w_k, i_k = _route(s_k, bias_f)
        w_parts.append(w_k)
        i_parts.append(i_k)
    wt = jnp.concatenate(w_parts, axis=1).T
    idx = jnp.concatenate(i_parts, axis=1).T
    return wt, idx


# speedup vs baseline: 1.4463x; 1.4463x over previous
import functools

import jax
import jax.numpy as jnp
from jax import lax
from jax.experimental import pallas as pl
from jax.experimental.pallas import tpu as pltpu
from jax.experimental.pallas import tpu_sc as plsc

_E = 64
_N_GROUPS = 8
_GS = _E // _N_GROUPS
_TOPK_GROUPS = 4
_TOPK = 8
_ROUTE_SCALE = 2.5

_NC = 2
_NS = 16
_NW = _NC * _NS
_L = 16


def _scores_body(x_ref, w_ref, b_ref, out_ref):
    logits = jax.lax.dot_general(
        w_ref[...], x_ref[...], dimension_numbers=(((1,), (1,)), ((), ())),
        preferred_element_type=jnp.float32)
    out_ref[...] = jax.nn.sigmoid(logits) + b_ref[...]


def _scores_t(x, w_pad, bias_col, t_half, blk_off):
    dim = x.shape[1]
    tt = 1024
    return pl.pallas_call(
        _scores_body,
        grid=(t_half // tt,),
        in_specs=[
            pl.BlockSpec((tt, dim), lambda i: (i + blk_off, 0)),
            pl.BlockSpec((_E, dim), lambda i: (0, 0)),
            pl.BlockSpec((_E, 1), lambda i: (0, 0)),
        ],
        out_specs=pl.BlockSpec((_E, tt), lambda i: (0, i)),
        out_shape=jax.ShapeDtypeStruct((_E, t_half), jnp.float32),
    )(x, w_pad, bias_col)


def _tree(op, xs):
    xs = list(xs)
    while len(xs) > 1:
        nxt = [op(xs[i], xs[i + 1]) for i in range(0, len(xs) - 1, 2)]
        if len(xs) % 2:
            nxt.append(xs[-1])
        xs = nxt
    return xs[0]


def _top2_sum(vs):
    neg = jnp.full((_L,), -jnp.inf, jnp.float32)
    pairs = [(x, neg) for x in vs]
    while len(pairs) > 1:
        nxt = []
        for i in range(0, len(pairs) - 1, 2):
            (a1, a2), (b1, b2) = pairs[i], pairs[i + 1]
            m1 = jnp.maximum(a1, b1)
            m2 = jnp.maximum(jnp.minimum(a1, b1), jnp.maximum(a2, b2))
            nxt.append((m1, m2))
        if len(pairs) % 2:
            nxt.append(pairs[-1])
        pairs = nxt
    m1, m2 = pairs[0]
    return m1 + m2


def _route_chunk(v, bias_ref):
    neg = jnp.full((_L,), -jnp.inf, jnp.float32)
    one = jnp.full((_L,), 1.0, jnp.float32)
    zero = jnp.full((_L,), 0.0, jnp.float32)

    t2 = [_top2_sum(v[g * _GS:(g + 1) * _GS]) for g in range(_N_GROUPS)]

    sel = []
    for g in range(_N_GROUPS):
        terms = []
        for h in range(_N_GROUPS):
            if h == g:
                continue
            c = (t2[h] >= t2[g]) if h < g else (t2[h] > t2[g])
            terms.append(jnp.where(c, one, zero))
        rk = _tree(jnp.add, terms)
        sel.append(rk < float(_TOPK_GROUPS))

    sm = [jnp.where(sel[e // _GS], v[e], neg) for e in range(_E)]
    e_const = [jnp.full((_L,), e, jnp.int32) for e in range(_E)]
    big = jnp.full((_L,), _E, jnp.int32)

    widx, wraw = [], []
    for r in range(_TOPK):
        m = _tree(jnp.maximum, sm)
        cand = [jnp.where(sm[e] == m, e_const[e], big) for e in range(_E)]
        idx = _tree(jnp.minimum, cand)
        bias_at = plsc.load_gather(bias_ref, [idx])
        widx.append(idx)
        wraw.append(m - bias_at)
        if r < _TOPK - 1:
            sm = [jnp.where(idx == e_const[e], neg, sm[e]) for e in range(_E)]

    wsum = _tree(jnp.add, wraw)
    wvals = [(w / wsum) * _ROUTE_SCALE for w in wraw]
    return wvals, widx


def _route_body(sT, bias_hbm, wout, iout,
                span_v, wspan_v, ispan_v, bias_v,
                *, span):
    wid = lax.axis_index("s") * _NC + lax.axis_index("c")
    base = wid * span
    pltpu.sync_copy(bias_hbm, bias_v)
    pltpu.sync_copy(sT.at[:, pl.ds(base, span)], span_v)
    def chunk(c, carry):
        off = c * _L
        v = [span_v[e, pl.ds(off, _L)] for e in range(_E)]
        wvals, ivals = _route_chunk(v, bias_v)
        for r in range(_TOPK):
            wspan_v[r, pl.ds(off, _L)] = wvals[r]
            ispan_v[r, pl.ds(off, _L)] = ivals[r]
        return carry

    lax.fori_loop(0, span // _L, chunk, 0)
    pltpu.sync_copy(wspan_v, wout.at[:, pl.ds(base, span)])
    pltpu.sync_copy(ispan_v, iout.at[:, pl.ds(base, span)])


def _route(s_t, bias):
    t = s_t.shape[1]
    span = t // _NW
    mesh = plsc.VectorSubcoreMesh(
        core_axis_name="c", subcore_axis_name="s",
        num_cores=_NC, num_subcores=_NS)
    fn = pl.kernel(
        functools.partial(_route_body, span=span),
        out_type=[
            jax.ShapeDtypeStruct((_TOPK, t), jnp.float32),
            jax.ShapeDtypeStruct((_TOPK, t), jnp.int32),
        ],
        mesh=mesh,
        compiler_params=pltpu.CompilerParams(
            use_tc_tiling_on_sc=False, needs_layout_passes=False),
        scratch_types=[
            pltpu.VMEM((_E, span), jnp.float32),
            pltpu.VMEM((_TOPK, span), jnp.float32),
            pltpu.VMEM((_TOPK, span), jnp.int32),
            pltpu.VMEM((_E,), jnp.float32),
        ],
    )
    return fn(s_t, bias)


_N_SPLITS = 2


@jax.jit
def kernel(x, weight, bias):
    t = x.shape[0]
    e = weight.shape[0]
    w_pad = jnp.pad(weight, ((0, 0), (1, 0)))
    bias_col = bias.reshape(e, 1).astype(jnp.float32)
    bias_f = bias.astype(jnp.float32)

    t_half = t // _N_SPLITS
    tt = 1024
    w_parts, i_parts = [], []
    for k in range(_N_SPLITS):
        s_k = _scores_t(x, w_pad, bias_col, t_half, k * (t_half // tt))
        w_k, i_k = _route(s_k, bias_f)
        w_parts.append(w_k)
        i_parts.append(i_k)
    wt = jnp.concatenate(w_parts, axis=1).T
    idx = jnp.concatenate(i_parts, axis=1).T
    return wt, idx
